# baseline (speedup 1.0000x reference)
import functools

import jax
import jax.numpy as jnp
from jax import lax
from jax.experimental import pallas as pl
from jax.experimental.pallas import tpu as pltpu
from jax.experimental.pallas import tpu_sc as plsc

NG = 50
CUTOFF = 10.0
DIM = 128
ORDER = 5
_DIMS = [DIM // 2 ** i for i in range(ORDER)][::-1]

_NCORES = 2
_NSUB = 16
_LN2 = 0.6931471805599453
_BN = 2000
_BE = 2048

_SC_PARAMS = pltpu.CompilerParams(
    needs_layout_passes=False, use_tc_tiling_on_sc=False)


def _ssp(v):
    return jnp.maximum(v, 0.0) + jnp.log1p(jnp.exp(-jnp.abs(v))) - _LN2


def _pad32(c):
    return max(32, ((c + 31) // 32) * 32)


def _bs_rows(bn, c):
    return pl.BlockSpec((bn, c), lambda i: (i, 0))


def _bs_full(shape):
    nd = len(shape)
    return pl.BlockSpec(shape, lambda i, _nd=nd: (0,) * _nd)


def _make_d2(n_nodes, e_pad):
    e_per = e_pad // (_NCORES * _NSUB)
    k2 = 2048
    nblk = e_per // k2
    krow = k2 // 128
    mesh = plsc.VectorSubcoreMesh(core_axis_name="c", subcore_axis_name="s")

    @functools.partial(
        pl.kernel,
        out_type=jax.ShapeDtypeStruct((e_pad // 128, 128), jnp.float32),
        mesh=mesh,
        scratch_types=[
            pltpu.VMEM((n_nodes,), jnp.float32),
            pltpu.VMEM((n_nodes,), jnp.float32),
            pltpu.VMEM((n_nodes,), jnp.float32),
            pltpu.VMEM((krow, 128), jnp.int32),
            pltpu.VMEM((krow, 128), jnp.int32),
            pltpu.VMEM((krow, 128), jnp.float32),
        ],
        compiler_params=_SC_PARAMS,
    )
    def d2k(px_hbm, py_hbm, pz_hbm, si_hbm, di_hbm, out_hbm,
            px, py, pz, sidx, didx, d2b):
        c = lax.axis_index("c")
        s = lax.axis_index("s")
        wkr = s * _NCORES + c
        pltpu.sync_copy(px_hbm, px)
        pltpu.sync_copy(py_hbm, py)
        pltpu.sync_copy(pz_hbm, pz)
        base_rows0 = wkr * (e_per // 128)

        def blk(b, carry):
            brow = base_rows0 + b * krow
            pltpu.sync_copy(si_hbm.at[pl.ds(brow, krow)], sidx)
            pltpu.sync_copy(di_hbm.at[pl.ds(brow, krow)], didx)

            def grp(j, carry2):
                for t in range(8):
                    sl = pl.ds(t * 16, 16)
                    vs = sidx[j, sl]
                    vd = didx[j, sl]
                    dx = plsc.load_gather(px, [vs]) - plsc.load_gather(px, [vd])
                    dy = plsc.load_gather(py, [vs]) - plsc.load_gather(py, [vd])
                    dz = plsc.load_gather(pz, [vs]) - plsc.load_gather(pz, [vd])
                    d2b[j, sl] = dx * dx + dy * dy + dz * dz
                return carry2

            lax.fori_loop(0, krow, grp, 0)
            pltpu.sync_copy(d2b, out_hbm.at[pl.ds(brow, krow)])
            return carry

        lax.fori_loop(0, nblk, blk, 0)

    return d2k


def _make_gms(n_nodes, n_pad, e_pad, hc):
    e_per = e_pad // _NSUB
    nsteps = e_per // 128
    rows_per = n_pad // _NSUB
    zr = 32
    nz = rows_per // zr
    mesh = plsc.VectorSubcoreMesh(core_axis_name="c", subcore_axis_name="s")

    @functools.partial(
        pl.kernel,
        out_type=jax.ShapeDtypeStruct((2, n_pad, hc), jnp.float32),
        mesh=mesh,
        scratch_types=[
            pltpu.VMEM((nsteps, 128), jnp.int32),
            pltpu.VMEM((nsteps, 128), jnp.int32),
            pltpu.VMEM((2, 128, hc), jnp.float32),
            pltpu.VMEM((2, 128, hc), jnp.float32),
            pltpu.VMEM((zr, hc), jnp.float32),
            pltpu.VMEM_SHARED((n_pad, hc), jnp.float32),
            pltpu.SemaphoreType.DMA,
            pltpu.SemaphoreType.DMA,
        ],
        compiler_params=_SC_PARAMS,
    )
    def gms(h2_hbm, w2_hbm, si_hbm, di_hbm, out_hbm,
            sidx, didx, rows, wrows, zbuf, agg, gs0, gs1):
        c = lax.axis_index("c")
        s = lax.axis_index("s")
        coff = c * n_nodes
        gsems = (gs0, gs1)

        def zrow(i, carry):
            for j in range(hc // 16):
                zbuf[i, pl.ds(j * 16, 16)] = jnp.zeros((16,), jnp.float32)
            return carry

        lax.fori_loop(0, zr, zrow, 0)
        r0 = s * rows_per
        for t in range(nz):
            pltpu.sync_copy(zbuf, agg.at[pl.ds(r0 + t * zr, zr)])
        plsc.subcore_barrier()

        i0row = s * nsteps
        pltpu.sync_copy(si_hbm.at[pl.ds(i0row, nsteps)], sidx)
        pltpu.sync_copy(di_hbm.at[pl.ds(i0row, nsteps)], didx)

        def shift(j, carry):
            for t in range(8):
                sl = pl.ds(t * 16, 16)
                sidx[j, sl] = sidx[j, sl] + coff
            return carry

        lax.fori_loop(0, nsteps, shift, 0)

        wbase = c * e_pad + s * e_per

        def fire(i, buf):
            pltpu.async_copy(h2_hbm.at[sidx.at[i]], rows.at[buf], gsems[buf])
            pltpu.async_copy(w2_hbm.at[pl.ds(wbase + i * 128, 128)],
                             wrows.at[buf], gsems[buf])

        def consume(i, buf):
            pltpu.make_async_copy(h2_hbm.at[sidx.at[i]], rows.at[buf],
                                  gsems[buf]).wait()
            pltpu.make_async_copy(w2_hbm.at[pl.ds(wbase + i * 128, 128)],
                                  wrows.at[buf], gsems[buf]).wait()
            rv = rows.at[buf]
            wv = wrows.at[buf]

            def mulrow(r, carry):
                for j in range(hc // 16):
                    sl = pl.ds(j * 16, 16)
                    rv[r, sl] = rv[r, sl] * wv[r, sl]
                return carry

            lax.fori_loop(0, 128, mulrow, 0)
            pltpu.sync_copy(rv, agg.at[didx.at[i]], add=True)

        fire(0, 0)

        def pair(g, carry):
            i0 = 2 * g
            fire(i0 + 1, 1)
            consume(i0, 0)
            fire(i0 + 2, 0)
            consume(i0 + 1, 1)
            return carry

        lax.fori_loop(0, nsteps // 2 - 1, pair, 0)
        ilast = nsteps - 2
        fire(ilast + 1, 1)
        consume(ilast, 0)
        consume(ilast + 1, 1)

        plsc.subcore_barrier()
        pltpu.sync_copy(agg.at[pl.ds(r0, rows_per)],
                        out_hbm.at[c, pl.ds(r0, rows_per)])

    return gms


def _filter_w(d2c, wn1, bn1, wn2, bn2, n_edges, hc, ncalls):
    e_pad = d2c.shape[0]
    cp = wn1.shape[1]
    delta = CUTOFF / (NG - 1)
    coeff = -0.5 / delta ** 2

    def body(d2_ref, wn1_ref, bn1_ref, wn2_ref, bn2_ref, *outs):
        i = pl.program_id(0)
        d = jnp.sqrt(d2_ref[...] + 1e-12)
        offs = delta * lax.broadcasted_iota(jnp.int32, (1, NG), 1).astype(
            jnp.float32)
        diff = d - offs
        rbf = jnp.exp(coeff * (diff * diff))
        u = _ssp(jnp.dot(rbf, wn1_ref[...],
                         preferred_element_type=jnp.float32) + bn1_ref[...])
        w = _ssp(jnp.dot(u, wn2_ref[...],
                         preferred_element_type=jnp.float32) + bn2_ref[...])
        rows = i * _BE + lax.broadcasted_iota(jnp.int32, (_BE, 1), 0)
        w = jnp.where(rows < n_edges, w, 0.0)
        for t in range(ncalls):
            outs[t][0] = w[:, (2 * t) * hc:(2 * t + 1) * hc]
            outs[t][1] = w[:, (2 * t + 1) * hc:(2 * t + 2) * hc]

    return pl.pallas_call(
        body,
        grid=(e_pad // _BE,),
        in_specs=[
            pl.BlockSpec((_BE, 1), lambda i: (i, 0)),
            _bs_full((NG, cp)),
            _bs_full((1, cp)),
            _bs_full((cp, cp)),
            _bs_full((1, cp)),
        ],
        out_specs=[pl.BlockSpec((2, _BE, hc), lambda i: (0, i, 0))
                   for _ in range(ncalls)],
        out_shape=[jax.ShapeDtypeStruct((2, e_pad, hc), jnp.float32)
                   for _ in range(ncalls)],
    )(d2c, wn1, bn1, wn2, bn2)


def _t0(x3d, w1p, hc, ncalls):
    n = x3d.shape[0]
    cin = x3d.shape[1]

    def body(x_ref, w_ref, *outs):
        h = jnp.dot(x_ref[...], w_ref[...], preferred_element_type=jnp.float32)
        for t in range(ncalls):
            outs[t][0] = h[:, (2 * t) * hc:(2 * t + 1) * hc]
            outs[t][1] = h[:, (2 * t + 1) * hc:(2 * t + 2) * hc]

    return pl.pallas_call(
        body,
        grid=(n // _BN,),
        in_specs=[_bs_rows(_BN, cin), _bs_full(w1p.shape)],
        out_specs=[pl.BlockSpec((2, _BN, hc), lambda i: (0, i, 0))
                   for _ in range(ncalls)],
        out_shape=[jax.ShapeDtypeStruct((2, n, hc), jnp.float32)
                   for _ in range(ncalls)],
    )(x3d, w1p)


def _transition(aggs, hcp, w2ch, b2, w1n, hcn, ncn, n, *, pwa=None, dw=None,
                lohi=None, emit_pwa=False, emit_dw=False, final=False):
    na = len(aggs)
    sd = sum(_DIMS)

    def body(*refs):
        refs = list(refs)
        ar = refs[:2 * na]
        w2r = refs[2 * na:4 * na]
        pos = 4 * na
        b2r = refs[pos]
        pos += 1
        pwar = dwr = w1r = None
        if emit_dw:
            pwar = refs[pos]
            pos += 1
        if lohi is not None:
            dwr = refs[pos]
            pos += 1
        if not final:
            w1r = refs[pos]
            pos += 1
        outs = refs[pos:]
        acc = b2r[...]
        for t in range(2 * na):
            acc = acc + jnp.dot(ar[t][0], w2r[t][...],
                                preferred_element_type=jnp.float32)
        o = _ssp(acc)
        if final:
            outs[0][...] = o
            return
        if emit_pwa:
            outs[ncn][...] = o[:, : _DIMS[0]]
            hin = o[:, _DIMS[0]:]
        elif emit_dw:
            outs[ncn][...] = o
            hin = pwar[...] * o[:, : _DIMS[0]]
        else:
            lo, hi = lohi
            hin = o * dwr[...][:, lo:hi]
        hn = jnp.dot(hin, w1r[...], preferred_element_type=jnp.float32)
        for t in range(ncn):
            outs[t][0] = hn[:, (2 * t) * hcn:(2 * t + 1) * hcn]
            outs[t][1] = hn[:, (2 * t + 1) * hcn:(2 * t + 2) * hcn]

    in_specs = []
    args = []
    for a in aggs:
        in_specs.append(pl.BlockSpec((1, _BN, hcp), lambda i: (0, i, 0)))
        in_specs.append(pl.BlockSpec((1, _BN, hcp), lambda i: (1, i, 0)))
        args += [a, a]
    for wc in w2ch:
        in_specs.append(_bs_full(wc.shape))
        args.append(wc)
    in_specs.append(_bs_full(b2.shape))
    args.append(b2)
    if emit_dw:
        in_specs.append(_bs_rows(_BN, _DIMS[0]))
        args.append(pwa)
    if lohi is not None:
        in_specs.append(_bs_rows(_BN, sd))
        args.append(dw)
    if not final:
        in_specs.append(_bs_full(w1n.shape))
        args.append(w1n)

    out_specs = []
    out_shape = []
    if final:
        cout = w2ch[0].shape[1]
        out_specs.append(_bs_rows(_BN, cout))
        out_shape.append(jax.ShapeDtypeStruct((n, cout), jnp.float32))
    else:
        for _ in range(ncn):
            out_specs.append(pl.BlockSpec((2, _BN, hcn), lambda i: (0, i, 0)))
            out_shape.append(jax.ShapeDtypeStruct((2, n, hcn), jnp.float32))
        if emit_pwa:
            out_specs.append(_bs_rows(_BN, _DIMS[0]))
            out_shape.append(jax.ShapeDtypeStruct((n, _DIMS[0]), jnp.float32))
        elif emit_dw:
            out_specs.append(_bs_rows(_BN, sd))
            out_shape.append(jax.ShapeDtypeStruct((n, sd), jnp.float32))

    res = pl.pallas_call(
        body,
        grid=(n // _BN,),
        in_specs=in_specs,
        out_specs=out_specs,
        out_shape=out_shape,
    )(*args)
    if final:
        return res[0]
    return list(res)


def _prep_filter(lp, cout):
    cp = _pad32(cout)
    wn1 = jnp.pad(lp["Wn1"], ((0, 0), (0, cp - cout)))
    bn1 = jnp.pad(lp["bn1"], (0, cp - cout)).reshape(1, cp)
    wn2 = jnp.pad(lp["Wn2"], ((0, cp - cout), (0, cp - cout)))
    bn2 = jnp.pad(lp["bn2"], (0, cp - cout)).reshape(1, cp)
    return wn1, bn1, wn2, bn2


def _prep_out(lp, cout, hc, ncalls):
    cp = _pad32(cout)
    w2p = jnp.pad(lp["W2"], ((0, cp - cout), (0, 0)))
    chunks = [w2p[i * hc:(i + 1) * hc] for i in range(2 * ncalls)]
    return chunks, lp["b2"].reshape(1, cout)


def kernel(x, x_3d, pos, edge_index, params):
    n = x_3d.shape[0]
    e = edge_index.shape[1]
    e_pad = ((e + 32767) // 32768) * 32768
    n_pad = ((n + 2047) // 2048) * 2048

    ei = jnp.pad(edge_index, ((0, 0), (0, e_pad - e)))
    si = ei[0].reshape(e_pad // 128, 128)
    di = ei[1].reshape(e_pad // 128, 128)
    d2 = _make_d2(n, e_pad)(pos[:, 0], pos[:, 1], pos[:, 2], si, di)
    d2c = d2.reshape(e_pad, 1)

    names = ["proj_in", "dwconv", "pw0", "pw1", "pw2", "pw3", "proj_out"]
    couts = [2 * DIM, sum(_DIMS)] + [_DIMS[i + 1] for i in range(ORDER - 1)] + [DIM]
    cps = [_pad32(c) for c in couts]
    hcs = [min(64, cp // 2) for cp in cps]
    ncs = [cp // (2 * hc) for cp, hc in zip(cps, hcs)]
    gms = {}
    for hc in set(hcs):
        gms[hc] = _make_gms(n, n_pad, e_pad, hc)

    wsp = []
    for li in range(7):
        fp = _prep_filter(params[names[li]], couts[li])
        wsp.append(_filter_w(d2c, *fp, e, hcs[li], ncs[li]))

    def run_edge(li, h2s):
        return [gms[hcs[li]](h2s[t].reshape(2 * n, hcs[li]),
                             wsp[li][t].reshape(2 * e_pad, hcs[li]), si, di)
                for t in range(ncs[li])]

    bounds = []
    start = 0
    for dcur in _DIMS:
        bounds.append((start, start + dcur))
        start += dcur

    p1 = params["proj_in"]
    h2s = _t0(x_3d, p1["W1"], hcs[0], ncs[0])
    aggs = run_edge(0, h2s)

    w2ch, b2 = _prep_out(p1, couts[0], hcs[0], ncs[0])
    pdw = params["dwconv"]
    w1dw = jnp.pad(pdw["W1"], ((0, 0), (0, cps[1] - couts[1])))
    *h2s, pwa = _transition(aggs, hcs[0], w2ch, b2, w1dw, hcs[1], ncs[1], n,
                            emit_pwa=True)
    aggs = run_edge(1, h2s)

    w2ch, b2 = _prep_out(pdw, couts[1], hcs[1], ncs[1])
    w1n = jnp.pad(params["pw0"]["W1"], ((0, 0), (0, cps[2] - couts[2])))
    *h2s, dw = _transition(aggs, hcs[1], w2ch, b2, w1n, hcs[2], ncs[2], n,
                           pwa=pwa, emit_dw=True)
    aggs = run_edge(2, h2s)

    for i in range(3, 7):
        w2ch, b2 = _prep_out(params[names[i - 1]], couts[i - 1],
                             hcs[i - 1], ncs[i - 1])
        w1n = params[names[i]]["W1"]
        if cps[i] != couts[i]:
            w1n = jnp.pad(w1n, ((0, 0), (0, cps[i] - couts[i])))
        h2s = _transition(aggs, hcs[i - 1], w2ch, b2, w1n, hcs[i], ncs[i], n,
                          dw=dw, lohi=bounds[i - 2])
        if not isinstance(h2s, (list, tuple)):
            h2s = [h2s]
        aggs = run_edge(i, h2s)

    w2ch, b2 = _prep_out(params["proj_out"], couts[6], hcs[6], ncs[6])
    return _transition(aggs, hcs[6], w2ch, b2, None, 0, 0, n, final=True)
